# baseline (speedup 1.0000x reference)
import functools

import jax
import jax.numpy as jnp
import numpy as np
from jax import lax
from jax.experimental import pallas as pl
from jax.experimental.pallas import tpu as pltpu
from jax.experimental.pallas import tpu_sc as plsc

NC = 2
NS = 16
NW = NC * NS
L = 16
LN_EPS = 1e-5


def _rsqrt_newton(x):
    i = lax.bitcast_convert_type(x, jnp.int32)
    i = jnp.int32(0x5F3759DF) - lax.shift_right_arithmetic(i, jnp.int32(1))
    y = lax.bitcast_convert_type(i, jnp.float32)
    half_x = jnp.float32(0.5) * x
    for _ in range(3):
        y = y * (jnp.float32(1.5) - half_x * y * y)
    return y


def _build(n_tot, S, chunk, emb, interpret=False):
    rows_pw = n_tot // NW
    G = rows_pw // chunk
    FV = emb // L
    inv_emb = np.float32(1.0 / emb)

    mesh = plsc.VectorSubcoreMesh(
        core_axis_name="c", subcore_axis_name="s", num_cores=NC, num_subcores=NS
    )

    @functools.partial(
        pl.kernel,
        out_type=jax.ShapeDtypeStruct((n_tot, emb), jnp.float32),
        mesh=mesh,
        scratch_types=[
            pltpu.VMEM((G, chunk), jnp.int32),
            pltpu.VMEM((S, emb), jnp.float32),
            pltpu.VMEM((2, emb), jnp.float32),
            pltpu.VMEM((2, chunk, emb), jnp.float32),
            pltpu.VMEM((2, chunk, emb), jnp.float32),
            pltpu.SemaphoreType.DMA,
            pltpu.SemaphoreType.DMA,
            pltpu.SemaphoreType.DMA,
            pltpu.SemaphoreType.DMA,
        ],
        compiler_params=pltpu.CompilerParams(
            needs_layout_passes=False, use_tc_tiling_on_sc=False
        ),
        interpret=interpret,
    )
    def k(idx_hbm, table_hbm, pos_hbm, wb_hbm, out_hbm,
          idx_v, pos_v, wb_v, in_v, out_v, gsem0, gsem1, osem0, osem1):
        wid = lax.axis_index("s") * NC + lax.axis_index("c")
        row0 = wid * rows_pw

        pltpu.sync_copy(idx_hbm.at[wid], idx_v)
        pltpu.sync_copy(pos_hbm, pos_v)
        pltpu.sync_copy(wb_hbm, wb_v)

        gsems = (gsem0, gsem1)
        osems = (osem0, osem1)

        def gather_start(g, b):
            pltpu.async_copy(table_hbm.at[idx_v.at[g]], in_v.at[b], gsems[b])

        def gather_wait(b):
            pltpu.make_async_copy(
                table_hbm.at[idx_v.at[0]], in_v.at[b], gsems[b]
            ).wait()

        def out_start(g, b):
            pltpu.async_copy(
                out_v.at[b], out_hbm.at[pl.ds(row0 + g * chunk, chunk)], osems[b]
            )

        def out_wait(b):
            pltpu.make_async_copy(
                out_v.at[b], out_hbm.at[pl.ds(0, chunk)], osems[b]
            ).wait()

        Ws = [wb_v[0, pl.ds(j * L, L)] for j in range(FV)]
        Bs = [wb_v[1, pl.ds(j * L, L)] for j in range(FV)]

        def compute_chunk(g, b):
            base_p = lax.rem(row0 + g * chunk, S)

            def row(i, p):
                xs = []
                for j in range(FV):
                    t = in_v[b, i, pl.ds(j * L, L)]
                    q = pos_v[p, pl.ds(j * L, L)]
                    xs.append(t + q)
                ssum = xs[0] + xs[1]
                qsum = xs[0] * xs[0] + xs[1] * xs[1]
                for j in range(2, FV):
                    ssum = ssum + xs[j]
                    qsum = qsum + xs[j] * xs[j]
                s = jnp.sum(ssum)
                q = jnp.sum(qsum)
                mean = s * inv_emb
                var = q * inv_emb - mean * mean
                rstd = _rsqrt_newton(var + np.float32(LN_EPS))
                mean_v = jnp.broadcast_to(mean, (L,))
                rstd_v = jnp.broadcast_to(rstd, (L,))
                for j in range(FV):
                    out_v[b, i, pl.ds(j * L, L)] = (
                        (xs[j] - mean_v) * rstd_v * Ws[j] + Bs[j]
                    )
                pn = p + 1
                return lax.select(pn == S, jnp.int32(0), pn)

            lax.fori_loop(0, chunk, row, base_p, unroll=4)

        gather_start(0, 0)
        if G > 1:
            gather_start(1, 1)

        def ring_step(outer, _):
            for b in range(2):
                g = outer * 2 + b

                @pl.when(g < G)
                def _():
                    gather_wait(b)

                    @pl.when(g >= 2)
                    def _():
                        out_wait(b)

                    compute_chunk(g, b)

                    @pl.when(g + 2 < G)
                    def _():
                        gather_start(g + 2, b)

                    out_start(g, b)
            return 0

        lax.fori_loop(0, (G + 1) // 2, ring_step, 0)

        out_wait(0)
        if G > 1:
            out_wait(1)

    return k


@functools.lru_cache(maxsize=None)
def _kernel_fn(n_tot, S, chunk, emb, interpret):
    return _build(n_tot, S, chunk, emb, interpret)


def _call(src, emb_table, pos_table, ln_w, ln_b, interpret=False):
    N, S = src.shape
    emb = emb_table.shape[1]
    n_tot = N * S
    rows_pw = n_tot // NW
    assert n_tot % NW == 0
    chunk = rows_pw
    for c in (128, 64, 32, 16, 8, 4, 2):
        if rows_pw % c == 0:
            chunk = c
            break
    G = rows_pw // chunk

    idx_r = src.reshape(NW, G, chunk).astype(jnp.int32)
    pos = pos_table[:S]
    wb = jnp.stack([ln_w, ln_b])
    fn = _kernel_fn(n_tot, S, chunk, emb, interpret)
    out = fn(idx_r, emb_table, pos, wb)
    return out.reshape(N, S, emb)


def kernel(src, emb_table, pos_table, ln_w, ln_b):
    return _call(src, emb_table, pos_table, ln_w, ln_b)
